# baseline (speedup 1.0000x reference)
import jax
import jax.numpy as jnp
from jax.experimental import pallas as pl
from jax.experimental.pallas import tpu as pltpu

_V = 431
_C = 1280
_S = 16
_G = 16


def _fused_body(ht_ref, w1t_ref, b1_ref, lnpw_ref, lnpb_ref, l1k_ref, rb1_ref,
                ln1w_ref, ln1b_ref, gk2_ref, bc_ref, adjt_ref,
                ln2w_ref, ln2b_ref, l2k_ref, rb2_ref, w3t_ref, b3_ref,
                out_ref, a2t_scr):
    @pl.when(pl.program_id(0) == 0)
    def _():
        a2t_scr[...] = jnp.dot(adjt_ref[...], adjt_ref[...],
                               preferred_element_type=jnp.float32)

    def ln_relu(v, f, w_ref, b_ref):
        v3 = v.reshape(_G, f, _V)
        u = jnp.mean(v3, axis=1, keepdims=True)
        s2 = jnp.mean((v3 - u) ** 2, axis=1, keepdims=True)
        t = (w_ref[...][None, :, :] * (v3 - u) * jax.lax.rsqrt(s2 + 1e-12)
             + b_ref[...][None, :, :])
        return jnp.maximum(t, 0.0).reshape(_G * f, _V)

    x1 = jnp.dot(ht_ref[...], w1t_ref[...],
                 preferred_element_type=jnp.float32) + b1_ref[...]
    t = ln_relu(x1, _S, lnpw_ref, lnpb_ref)
    y = jnp.dot(l1k_ref[...], t, preferred_element_type=jnp.float32) + rb1_ref[...]
    y = ln_relu(y, 8, ln1w_ref, ln1b_ref)
    q = jnp.dot(gk2_ref[...], y, preferred_element_type=jnp.float32)
    y = jnp.dot(q, a2t_scr[...], preferred_element_type=jnp.float32) + bc_ref[...]
    t2 = ln_relu(y, 8, ln2w_ref, ln2b_ref)
    y2 = jnp.dot(l2k_ref[...], t2, preferred_element_type=jnp.float32) + rb2_ref[...]
    z = x1 + y2
    out_ref[...] = jnp.dot(z, w3t_ref[...],
                           preferred_element_type=jnp.float32) + b3_ref[...]


def kernel(hidden_states, W1, b1, ln_pre_w, ln_pre_b, lin1_w, lin1_b,
           ln1_w, ln1_b, gcn_w, gcn_b, adjmat, ln2_w, ln2_b,
           lin2_w, lin2_b, W3, b3):
    T = hidden_states.shape[2]
    hsr = hidden_states.reshape(-1, _C, _S)
    n = hsr.shape[0]
    ng = n // _G
    ht = hsr.transpose(0, 2, 1).reshape(n * _S, _C)

    eye = jnp.eye(_G, dtype=jnp.float32)
    l1k = jnp.kron(eye, lin1_w)
    gk2 = jnp.kron(eye, (gcn_w @ gcn_w).T)
    l2k = jnp.kron(eye, lin2_w)
    rb1 = jnp.tile(lin1_b, _G)[:, None]
    rb2 = jnp.tile(lin2_b, _G)[:, None]
    r = adjmat.sum(axis=1)
    bct = (gcn_b @ gcn_w)[:, None] * r[None, :] + gcn_b[:, None]
    bc = jnp.tile(bct, (_G, 1))

    const = lambda i: (0, 0)
    out = pl.pallas_call(
        _fused_body,
        grid=(ng,),
        in_specs=[
            pl.BlockSpec((_G * _S, _C), lambda i: (i, 0)),
            pl.BlockSpec((_C, _V), const),
            pl.BlockSpec((1, _V), const),
            pl.BlockSpec((_S, 1), const),
            pl.BlockSpec((_S, 1), const),
            pl.BlockSpec((_G * 8, _G * _S), const),
            pl.BlockSpec((_G * 8, 1), const),
            pl.BlockSpec((8, 1), const),
            pl.BlockSpec((8, 1), const),
            pl.BlockSpec((_G * 8, _G * 8), const),
            pl.BlockSpec((_G * 8, _V), const),
            pl.BlockSpec((_V, _V), const),
            pl.BlockSpec((8, 1), const),
            pl.BlockSpec((8, 1), const),
            pl.BlockSpec((_G * _S, _G * 8), const),
            pl.BlockSpec((_G * _S, 1), const),
            pl.BlockSpec((_V, _C), const),
            pl.BlockSpec((1, _C), const),
        ],
        out_specs=pl.BlockSpec((_G * _S, _C), lambda i: (i, 0)),
        out_shape=jax.ShapeDtypeStruct((n * _S, _C), jnp.float32),
        scratch_shapes=[pltpu.VMEM((_V, _V), jnp.float32)],
    )(ht, W1.T, b1[None, :], ln_pre_w[:, None], ln_pre_b[:, None],
      l1k, rb1, ln1_w[:, None], ln1_b[:, None], gk2, bc, adjmat.T,
      ln2_w[:, None], ln2_b[:, None], l2k, rb2, W3.T, b3[None, :])

    out = out.reshape(n, _S, _C).transpose(0, 2, 1)
    return out.reshape(-1, _C, T, 4, 4)
